# baseline (speedup 1.0000x reference)
import functools
import jax
import jax.numpy as jnp
from jax import lax
from jax.experimental import pallas as pl
from jax.experimental.pallas import tpu as pltpu
from jax.experimental.pallas import tpu_sc as plsc

_TOKENS = 32768
_E = 64
_TAU = 0.25
_NW = 32
_TPW = _TOKENS // _NW
_C = 256
_NCHUNK = _TPW // _C


def _rev(v):
    return lax.rev(v, (0,))


def _sortd(v):
    k, _ = plsc.sort_key_val(v, v, descending=True)
    return k


def _sort64_desc(x0, x1, x2, x3):
    s0, s1, s2, s3 = _sortd(x0), _sortd(x1), _sortd(x2), _sortd(x3)
    rb = _rev(s1)
    u1, v1 = _sortd(jnp.maximum(s0, rb)), _sortd(jnp.minimum(s0, rb))
    rb = _rev(s3)
    u2, v2 = _sortd(jnp.maximum(s2, rb)), _sortd(jnp.minimum(s2, rb))
    a, b = _rev(v2), _rev(u2)
    p, r = jnp.maximum(u1, a), jnp.minimum(u1, a)
    q, s = jnp.maximum(v1, b), jnp.minimum(v1, b)
    d0 = _sortd(jnp.maximum(p, q))
    d1 = _sortd(jnp.minimum(p, q))
    d2 = _sortd(jnp.maximum(r, s))
    d3 = _sortd(jnp.minimum(r, s))
    return d0, d1, d2, d3


def _popcnt(b):
    return plsc.all_reduce_population_count(b)


def _first_eq_mask(x, vstar):
    eq = [xi == vstar for xi in x]
    n0 = _popcnt(eq[0])
    n1 = n0 + _popcnt(eq[1])
    n2 = n1 + _popcnt(eq[2])
    cc0 = plsc.cumsum(eq[0].astype(jnp.int32))
    cc1 = plsc.cumsum(eq[1].astype(jnp.int32)) + n0
    cc2 = plsc.cumsum(eq[2].astype(jnp.int32)) + n1
    cc3 = plsc.cumsum(eq[3].astype(jnp.int32)) + n2
    return eq, (cc0, cc1, cc2, cc3)


def _slow_masks(x0, x1, x2, x3):
    d0, d1, d2, d3 = _sort64_desc(x0, x1, x2, x3)
    r0 = jnp.sum(d0)
    r1 = r0 + jnp.sum(d1)
    r2 = r1 + jnp.sum(d2)
    c0 = plsc.cumsum(d0)
    c1 = plsc.cumsum(d1) + r0
    c2 = plsc.cumsum(d2) + r1
    c3 = plsc.cumsum(d3) + r2
    cnt = (_popcnt(c0 < _TAU) + _popcnt(c1 < _TAU)
           + _popcnt(c2 < _TAU) + _popcnt(c3 < _TAU))
    k = jnp.minimum(cnt + 1, _E)
    km1 = k - 1
    iot = lax.iota(jnp.int32, 16)
    pick = (jnp.where(iot == km1, d0, 0.0)
            + jnp.where(iot + 16 == km1, d1, 0.0)
            + jnp.where(iot + 32 == km1, d2, 0.0)
            + jnp.where(iot + 48 == km1, d3, 0.0))
    vstar = jnp.zeros((16,), jnp.float32) + jnp.sum(pick)
    gt = [x0 > vstar, x1 > vstar, x2 > vstar, x3 > vstar]
    g = _popcnt(gt[0]) + _popcnt(gt[1]) + _popcnt(gt[2]) + _popcnt(gt[3])
    mm = k - g
    eq, cc = _first_eq_mask((x0, x1, x2, x3), vstar)
    return tuple(gti | (e & (c <= mm))
                 for gti, e, c in zip(gt, eq, cc))


def _sc_body(ntok, x_hbm, mask_hbm, routed_hbm, x_v, mask_v, routed_v):
    wid = lax.axis_index("s") * 2 + lax.axis_index("c")
    tpw = ntok // _NW
    base = wid * tpw
    nchunk = tpw // _C

    def chunk(i, carry):
        tok0 = base + i * _C
        pltpu.sync_copy(x_hbm.at[pl.ds(tok0, _C)], x_v)

        iot = lax.iota(jnp.int32, 16)
        iots = (iot, iot + 16, iot + 32, iot + 48)

        def load_token(t):
            return (x_v[t, 0:16], x_v[t, 16:32],
                    x_v[t, 32:48], x_v[t, 48:64])

        def store(t, xs, sels):
            x0, x1, x2, x3 = xs
            s0, s1, s2, s3 = sels
            mask_v[t, 0:16] = s0.astype(jnp.int32)
            mask_v[t, 16:32] = s1.astype(jnp.int32)
            mask_v[t, 32:48] = s2.astype(jnp.int32)
            mask_v[t, 48:64] = s3.astype(jnp.int32)
            routed_v[t, 0:16] = jnp.where(s0, x0, 0.0)
            routed_v[t, 16:32] = jnp.where(s1, x1, 0.0)
            routed_v[t, 32:48] = jnp.where(s2, x2, 0.0)
            routed_v[t, 48:64] = jnp.where(s3, x3, 0.0)

        @plsc.parallel_loop(0, _C, unroll=4, carry=jnp.float32(1.0))
        def tok(t, mn):
            x0, x1, x2, x3 = load_token(t)
            mx = jnp.max(jnp.maximum(jnp.maximum(x0, x1),
                                     jnp.maximum(x2, x3)))
            mxv = jnp.zeros((16,), jnp.float32) + mx
            pos = jnp.minimum(
                jnp.minimum(jnp.where(x0 == mxv, iots[0], _E),
                            jnp.where(x1 == mxv, iots[1], _E)),
                jnp.minimum(jnp.where(x2 == mxv, iots[2], _E),
                            jnp.where(x3 == mxv, iots[3], _E)))
            fidx = jnp.zeros((16,), jnp.int32) + jnp.min(pos)
            store(t, (x0, x1, x2, x3), tuple(io == fidx for io in iots))
            return jnp.minimum(mn, mx)

        @pl.when(tok < _TAU)
        def _():
            def fix(t, carry):
                x0, x1, x2, x3 = load_token(t)
                mx = jnp.max(jnp.maximum(jnp.maximum(x0, x1),
                                         jnp.maximum(x2, x3)))

                @pl.when(mx < _TAU)
                def _():
                    store(t, (x0, x1, x2, x3),
                          _slow_masks(x0, x1, x2, x3))

                return carry

            lax.fori_loop(0, _C, fix, 0)

        pltpu.sync_copy(mask_v, mask_hbm.at[pl.ds(tok0, _C)])
        pltpu.sync_copy(routed_v, routed_hbm.at[pl.ds(tok0, _C)])
        return carry

    lax.fori_loop(0, nchunk, chunk, 0)


def _make_sc_kernel(ntok):
    return functools.partial(
        pl.kernel,
        out_type=[
            jax.ShapeDtypeStruct((ntok, _E), jnp.int32),
            jax.ShapeDtypeStruct((ntok, _E), jnp.float32),
        ],
        mesh=plsc.VectorSubcoreMesh(core_axis_name="c", subcore_axis_name="s"),
        scratch_types=[
            pltpu.VMEM((_C, _E), jnp.float32),
            pltpu.VMEM((_C, _E), jnp.int32),
            pltpu.VMEM((_C, _E), jnp.float32),
        ],
        compiler_params=pltpu.CompilerParams(needs_layout_passes=False),
    )(functools.partial(_sc_body, ntok))


_NSPLIT = 2
_sc_kernel_half = _make_sc_kernel(_TOKENS // _NSPLIT)


def kernel(routing_tensor):
    h = _TOKENS // _NSPLIT
    parts = [_sc_kernel_half(routing_tensor[i * h:(i + 1) * h])
             for i in range(_NSPLIT)]
    mask = jnp.concatenate([p[0] for p in parts], axis=0)
    routed = jnp.concatenate([p[1] for p in parts], axis=0)
    return (mask, routed)


# speedup vs baseline: 1.1916x; 1.1916x over previous
import functools
import jax
import jax.numpy as jnp
from jax import lax
from jax.experimental import pallas as pl
from jax.experimental.pallas import tpu as pltpu
from jax.experimental.pallas import tpu_sc as plsc

_TOKENS = 32768
_E = 64
_TAU = 0.25
_NW = 32
_TPW = _TOKENS // _NW
_C = 256
_NCHUNK = _TPW // _C


def _rev(v):
    return lax.rev(v, (0,))


def _sortd(v):
    k, _ = plsc.sort_key_val(v, v, descending=True)
    return k


def _sort64_desc(x0, x1, x2, x3):
    s0, s1, s2, s3 = _sortd(x0), _sortd(x1), _sortd(x2), _sortd(x3)
    rb = _rev(s1)
    u1, v1 = _sortd(jnp.maximum(s0, rb)), _sortd(jnp.minimum(s0, rb))
    rb = _rev(s3)
    u2, v2 = _sortd(jnp.maximum(s2, rb)), _sortd(jnp.minimum(s2, rb))
    a, b = _rev(v2), _rev(u2)
    p, r = jnp.maximum(u1, a), jnp.minimum(u1, a)
    q, s = jnp.maximum(v1, b), jnp.minimum(v1, b)
    d0 = _sortd(jnp.maximum(p, q))
    d1 = _sortd(jnp.minimum(p, q))
    d2 = _sortd(jnp.maximum(r, s))
    d3 = _sortd(jnp.minimum(r, s))
    return d0, d1, d2, d3


def _popcnt(b):
    return plsc.all_reduce_population_count(b)


def _first_eq_mask(x, vstar):
    eq = [xi == vstar for xi in x]
    n0 = _popcnt(eq[0])
    n1 = n0 + _popcnt(eq[1])
    n2 = n1 + _popcnt(eq[2])
    cc0 = plsc.cumsum(eq[0].astype(jnp.int32))
    cc1 = plsc.cumsum(eq[1].astype(jnp.int32)) + n0
    cc2 = plsc.cumsum(eq[2].astype(jnp.int32)) + n1
    cc3 = plsc.cumsum(eq[3].astype(jnp.int32)) + n2
    return eq, (cc0, cc1, cc2, cc3)


def _slow_masks(x0, x1, x2, x3):
    d0, d1, d2, d3 = _sort64_desc(x0, x1, x2, x3)
    r0 = jnp.sum(d0)
    r1 = r0 + jnp.sum(d1)
    r2 = r1 + jnp.sum(d2)
    c0 = plsc.cumsum(d0)
    c1 = plsc.cumsum(d1) + r0
    c2 = plsc.cumsum(d2) + r1
    c3 = plsc.cumsum(d3) + r2
    cnt = (_popcnt(c0 < _TAU) + _popcnt(c1 < _TAU)
           + _popcnt(c2 < _TAU) + _popcnt(c3 < _TAU))
    k = jnp.minimum(cnt + 1, _E)
    km1 = k - 1
    iot = lax.iota(jnp.int32, 16)
    pick = (jnp.where(iot == km1, d0, 0.0)
            + jnp.where(iot + 16 == km1, d1, 0.0)
            + jnp.where(iot + 32 == km1, d2, 0.0)
            + jnp.where(iot + 48 == km1, d3, 0.0))
    vstar = jnp.zeros((16,), jnp.float32) + jnp.sum(pick)
    gt = [x0 > vstar, x1 > vstar, x2 > vstar, x3 > vstar]
    g = _popcnt(gt[0]) + _popcnt(gt[1]) + _popcnt(gt[2]) + _popcnt(gt[3])
    mm = k - g
    eq, cc = _first_eq_mask((x0, x1, x2, x3), vstar)
    return tuple(gti | (e & (c <= mm))
                 for gti, e, c in zip(gt, eq, cc))


def _sc_body(x_hbm, mask_hbm, routed_hbm, x_v, mask_v, routed_v):
    wid = lax.axis_index("s") * 2 + lax.axis_index("c")
    base = wid * _TPW

    def chunk(i, carry):
        tok0 = base + i * _C
        pltpu.sync_copy(x_hbm.at[pl.ds(tok0, _C)], x_v)

        iot = lax.iota(jnp.int32, 16)
        iots = (iot, iot + 16, iot + 32, iot + 48)

        def load_token(t):
            return (x_v[t, 0:16], x_v[t, 16:32],
                    x_v[t, 32:48], x_v[t, 48:64])

        def store(t, xs, sels):
            x0, x1, x2, x3 = xs
            s0, s1, s2, s3 = sels
            mask_v[t, 0:16] = s0.astype(jnp.int32)
            mask_v[t, 16:32] = s1.astype(jnp.int32)
            mask_v[t, 32:48] = s2.astype(jnp.int32)
            mask_v[t, 48:64] = s3.astype(jnp.int32)
            routed_v[t, 0:16] = jnp.where(s0, x0, 0.0)
            routed_v[t, 16:32] = jnp.where(s1, x1, 0.0)
            routed_v[t, 32:48] = jnp.where(s2, x2, 0.0)
            routed_v[t, 48:64] = jnp.where(s3, x3, 0.0)

        @plsc.parallel_loop(0, _C, unroll=8, carry=jnp.float32(1.0))
        def tok(t, mn):
            x0, x1, x2, x3 = load_token(t)
            mx = jnp.max(jnp.maximum(jnp.maximum(x0, x1),
                                     jnp.maximum(x2, x3)))
            mxv = jnp.zeros((16,), jnp.float32) + mx
            pos = jnp.minimum(
                jnp.minimum(jnp.where(x0 == mxv, iots[0], _E),
                            jnp.where(x1 == mxv, iots[1], _E)),
                jnp.minimum(jnp.where(x2 == mxv, iots[2], _E),
                            jnp.where(x3 == mxv, iots[3], _E)))
            fidx = jnp.zeros((16,), jnp.int32) + jnp.min(pos)
            store(t, (x0, x1, x2, x3), tuple(io == fidx for io in iots))
            return jnp.minimum(mn, mx)

        @pl.when(tok < _TAU)
        def _():
            def fix(t, carry):
                x0, x1, x2, x3 = load_token(t)
                mx = jnp.max(jnp.maximum(jnp.maximum(x0, x1),
                                         jnp.maximum(x2, x3)))

                @pl.when(mx < _TAU)
                def _():
                    store(t, (x0, x1, x2, x3),
                          _slow_masks(x0, x1, x2, x3))

                return carry

            lax.fori_loop(0, _C, fix, 0)

        pltpu.sync_copy(mask_v, mask_hbm.at[pl.ds(tok0, _C)])
        pltpu.sync_copy(routed_v, routed_hbm.at[pl.ds(tok0, _C)])
        return carry

    lax.fori_loop(0, _NCHUNK, chunk, 0)


_sc_kernel = functools.partial(
    pl.kernel,
    out_type=[
        jax.ShapeDtypeStruct((_TOKENS, _E), jnp.int32),
        jax.ShapeDtypeStruct((_TOKENS, _E), jnp.float32),
    ],
    mesh=plsc.VectorSubcoreMesh(core_axis_name="c", subcore_axis_name="s"),
    scratch_types=[
        pltpu.VMEM((_C, _E), jnp.float32),
        pltpu.VMEM((_C, _E), jnp.int32),
        pltpu.VMEM((_C, _E), jnp.float32),
    ],
    compiler_params=pltpu.CompilerParams(needs_layout_passes=False),
)(_sc_body)


def kernel(routing_tensor):
    mask, routed = _sc_kernel(routing_tensor)
    return (mask, routed)


# speedup vs baseline: 1.3750x; 1.1539x over previous
import functools
import jax
import jax.numpy as jnp
from jax import lax
from jax.experimental import pallas as pl
from jax.experimental.pallas import tpu as pltpu
from jax.experimental.pallas import tpu_sc as plsc

_TOKENS = 32768
_E = 64
_TAU = 0.25
_NW = 32
_TPW = _TOKENS // _NW
_C = 128
_NCHUNK = _TPW // _C


def _rev(v):
    return lax.rev(v, (0,))


def _sortd(v):
    k, _ = plsc.sort_key_val(v, v, descending=True)
    return k


def _sort64_desc(x0, x1, x2, x3):
    s0, s1, s2, s3 = _sortd(x0), _sortd(x1), _sortd(x2), _sortd(x3)
    rb = _rev(s1)
    u1, v1 = _sortd(jnp.maximum(s0, rb)), _sortd(jnp.minimum(s0, rb))
    rb = _rev(s3)
    u2, v2 = _sortd(jnp.maximum(s2, rb)), _sortd(jnp.minimum(s2, rb))
    a, b = _rev(v2), _rev(u2)
    p, r = jnp.maximum(u1, a), jnp.minimum(u1, a)
    q, s = jnp.maximum(v1, b), jnp.minimum(v1, b)
    d0 = _sortd(jnp.maximum(p, q))
    d1 = _sortd(jnp.minimum(p, q))
    d2 = _sortd(jnp.maximum(r, s))
    d3 = _sortd(jnp.minimum(r, s))
    return d0, d1, d2, d3


def _popcnt(b):
    return plsc.all_reduce_population_count(b)


def _first_eq_mask(x, vstar):
    eq = [xi == vstar for xi in x]
    n0 = _popcnt(eq[0])
    n1 = n0 + _popcnt(eq[1])
    n2 = n1 + _popcnt(eq[2])
    cc0 = plsc.cumsum(eq[0].astype(jnp.int32))
    cc1 = plsc.cumsum(eq[1].astype(jnp.int32)) + n0
    cc2 = plsc.cumsum(eq[2].astype(jnp.int32)) + n1
    cc3 = plsc.cumsum(eq[3].astype(jnp.int32)) + n2
    return eq, (cc0, cc1, cc2, cc3)


def _slow_masks(x0, x1, x2, x3):
    d0, d1, d2, d3 = _sort64_desc(x0, x1, x2, x3)
    r0 = jnp.sum(d0)
    r1 = r0 + jnp.sum(d1)
    r2 = r1 + jnp.sum(d2)
    c0 = plsc.cumsum(d0)
    c1 = plsc.cumsum(d1) + r0
    c2 = plsc.cumsum(d2) + r1
    c3 = plsc.cumsum(d3) + r2
    cnt = (_popcnt(c0 < _TAU) + _popcnt(c1 < _TAU)
           + _popcnt(c2 < _TAU) + _popcnt(c3 < _TAU))
    k = jnp.minimum(cnt + 1, _E)
    km1 = k - 1
    iot = lax.iota(jnp.int32, 16)
    pick = (jnp.where(iot == km1, d0, 0.0)
            + jnp.where(iot + 16 == km1, d1, 0.0)
            + jnp.where(iot + 32 == km1, d2, 0.0)
            + jnp.where(iot + 48 == km1, d3, 0.0))
    vstar = jnp.zeros((16,), jnp.float32) + jnp.sum(pick)
    gt = [x0 > vstar, x1 > vstar, x2 > vstar, x3 > vstar]
    g = _popcnt(gt[0]) + _popcnt(gt[1]) + _popcnt(gt[2]) + _popcnt(gt[3])
    mm = k - g
    eq, cc = _first_eq_mask((x0, x1, x2, x3), vstar)
    return tuple(gti | (e & (c <= mm))
                 for gti, e, c in zip(gt, eq, cc))


def _process_chunk(b, x_v, mask_v, routed_v):
    iot = lax.iota(jnp.int32, 16)
    iots = (iot, iot + 16, iot + 32, iot + 48)

    def load_token(t):
        return (x_v[b, t, 0:16], x_v[b, t, 16:32],
                x_v[b, t, 32:48], x_v[b, t, 48:64])

    def store(t, xs, sels):
        x0, x1, x2, x3 = xs
        s0, s1, s2, s3 = sels
        mask_v[b, t, 0:16] = s0.astype(jnp.int32)
        mask_v[b, t, 16:32] = s1.astype(jnp.int32)
        mask_v[b, t, 32:48] = s2.astype(jnp.int32)
        mask_v[b, t, 48:64] = s3.astype(jnp.int32)
        routed_v[b, t, 0:16] = jnp.where(s0, x0, 0.0)
        routed_v[b, t, 16:32] = jnp.where(s1, x1, 0.0)
        routed_v[b, t, 32:48] = jnp.where(s2, x2, 0.0)
        routed_v[b, t, 48:64] = jnp.where(s3, x3, 0.0)

    @plsc.parallel_loop(0, _C, unroll=4, carry=jnp.float32(1.0))
    def tok(t, mn):
        x0, x1, x2, x3 = load_token(t)
        mx = jnp.max(jnp.maximum(jnp.maximum(x0, x1), jnp.maximum(x2, x3)))
        mxv = jnp.zeros((16,), jnp.float32) + mx
        pos = jnp.minimum(
            jnp.minimum(jnp.where(x0 == mxv, iots[0], _E),
                        jnp.where(x1 == mxv, iots[1], _E)),
            jnp.minimum(jnp.where(x2 == mxv, iots[2], _E),
                        jnp.where(x3 == mxv, iots[3], _E)))
        fidx = jnp.zeros((16,), jnp.int32) + jnp.min(pos)
        store(t, (x0, x1, x2, x3), tuple(io == fidx for io in iots))
        return jnp.minimum(mn, mx)

    @pl.when(tok < _TAU)
    def _():
        def fix(t, carry):
            x0, x1, x2, x3 = load_token(t)
            mx = jnp.max(jnp.maximum(jnp.maximum(x0, x1),
                                     jnp.maximum(x2, x3)))

            @pl.when(mx < _TAU)
            def _():
                store(t, (x0, x1, x2, x3), _slow_masks(x0, x1, x2, x3))

            return carry

        lax.fori_loop(0, _C, fix, 0)


def _sc_body(x_hbm, mask_hbm, routed_hbm, x_v, mask_v, routed_v,
             si0, si1, sm0, sm1, sr0, sr1):
    wid = lax.axis_index("s") * 2 + lax.axis_index("c")
    base = wid * _TPW
    sin = (si0, si1)
    smk = (sm0, sm1)
    srt = (sr0, sr1)

    def rows(c):
        return pl.ds(base + c * _C, _C)

    in_d = {0: pltpu.async_copy(x_hbm.at[rows(0)], x_v.at[0], sin[0])}
    out_d = {}
    for c in range(_NCHUNK):
        b = c % 2
        if c + 1 < _NCHUNK:
            nb = (c + 1) % 2
            in_d[c + 1] = pltpu.async_copy(
                x_hbm.at[rows(c + 1)], x_v.at[nb], sin[nb])
        in_d[c].wait()
        if c >= 2:
            out_d[c - 2][0].wait()
            out_d[c - 2][1].wait()
        _process_chunk(b, x_v, mask_v, routed_v)
        out_d[c] = (
            pltpu.async_copy(mask_v.at[b], mask_hbm.at[rows(c)], smk[b]),
            pltpu.async_copy(routed_v.at[b], routed_hbm.at[rows(c)], srt[b]),
        )
    out_d[_NCHUNK - 2][0].wait()
    out_d[_NCHUNK - 2][1].wait()
    out_d[_NCHUNK - 1][0].wait()
    out_d[_NCHUNK - 1][1].wait()


_sc_kernel = functools.partial(
    pl.kernel,
    out_type=[
        jax.ShapeDtypeStruct((_TOKENS, _E), jnp.int32),
        jax.ShapeDtypeStruct((_TOKENS, _E), jnp.float32),
    ],
    mesh=plsc.VectorSubcoreMesh(core_axis_name="c", subcore_axis_name="s"),
    scratch_types=[
        pltpu.VMEM((2, _C, _E), jnp.float32),
        pltpu.VMEM((2, _C, _E), jnp.int32),
        pltpu.VMEM((2, _C, _E), jnp.float32),
        pltpu.SemaphoreType.DMA,
        pltpu.SemaphoreType.DMA,
        pltpu.SemaphoreType.DMA,
        pltpu.SemaphoreType.DMA,
        pltpu.SemaphoreType.DMA,
        pltpu.SemaphoreType.DMA,
    ],
    compiler_params=pltpu.CompilerParams(needs_layout_passes=False),
)(_sc_body)


def kernel(routing_tensor):
    mask, routed = _sc_kernel(routing_tensor)
    return (mask, routed)
